# baseline (speedup 1.0000x reference)
import functools

import jax
import jax.numpy as jnp
from jax import lax
from jax.experimental import pallas as pl
from jax.experimental.pallas import tpu as pltpu
from jax.experimental.pallas import tpu_sc as plsc

B = 4096
D = 128
ROWS = 165
NBUF = 10

NC = 2
NS = 16
NW = NC * NS
GR = B // NW
PR = 2 * B // NW


def _tc_body(h_hbm, wx_ref, wy_ref, oh_hbm,
             inb, outb, pe_ref, in_sems, out_sems):
    i = pl.program_id(0)

    def in_copy(chunk, slot):
        return pltpu.make_async_copy(h_hbm.at[chunk], inb.at[slot],
                                     in_sems.at[slot])

    def out_copy(chunk, slot):
        return pltpu.make_async_copy(outb.at[slot], oh_hbm.at[chunk],
                                     out_sems.at[slot])

    @pl.when(i == 0)
    def _prime():
        wx = wx_ref[...]
        for y in range(11):
            pe_ref[pl.ds(15 * y, 15), :] = wy_ref[y:y + 1, :] + wx
        for k in range(NBUF):
            in_copy(k, k).start()

    s = jax.lax.rem(i, NBUF)
    in_copy(i, s).wait()

    @pl.when(i >= NBUF)
    def _wait_out():
        out_copy(i, s).wait()

    outb[s] = inb[s] + pe_ref[pl.ds(i, 1), :]
    out_copy(i, s).start()

    @pl.when(i + NBUF < ROWS)
    def _next_in():
        in_copy(i + NBUF, s).start()

    @pl.when(i == ROWS - 1)
    def _drain():
        for k in range(NBUF):
            out_copy(0, k).wait()


def _sc_body(g_hbm, p_hbm, pg_hbm, pp_hbm, og_hbm, op_hbm,
             gbuf, pbuf, pgbuf, ppbuf):
    wid = lax.axis_index("s") * NC + lax.axis_index("c")
    pltpu.sync_copy(pg_hbm, pgbuf)
    pltpu.sync_copy(pp_hbm, ppbuf)
    pgv = [pgbuf[pl.ds(16 * j, 16)] for j in range(8)]
    ppv = [ppbuf[pl.ds(16 * j, 16)] for j in range(16)]

    gbase = wid * GR
    pltpu.sync_copy(g_hbm.at[pl.ds(gbase, GR)], gbuf)

    def g_row(r, _):
        for j in range(8):
            sl = pl.ds(16 * j, 16)
            gbuf[r, sl] = gbuf[r, sl] + pgv[j]
        return _

    lax.fori_loop(0, GR, g_row, 0)
    pltpu.sync_copy(gbuf, og_hbm.at[pl.ds(gbase, GR)])

    pbase = wid * PR
    pltpu.sync_copy(p_hbm.at[pl.ds(pbase, PR)], pbuf)

    def p_rows(r2, _):
        for par in range(2):
            r = 2 * r2 + par
            for j in range(8):
                sl = pl.ds(16 * j, 16)
                pbuf[r, sl] = pbuf[r, sl] + ppv[8 * par + j]
        return _

    lax.fori_loop(0, PR // 2, p_rows, 0)
    pltpu.sync_copy(pbuf, op_hbm.at[pl.ds(pbase, PR)])


_sc_call = functools.partial(
    pl.kernel,
    out_type=[
        jax.ShapeDtypeStruct((B, D), jnp.float32),
        jax.ShapeDtypeStruct((2 * B, D), jnp.float32),
    ],
    mesh=plsc.VectorSubcoreMesh(core_axis_name="c", subcore_axis_name="s",
                                num_cores=NC, num_subcores=NS),
    scratch_types=[
        pltpu.VMEM((GR, D), jnp.float32),
        pltpu.VMEM((PR, D), jnp.float32),
        pltpu.VMEM((D,), jnp.float32),
        pltpu.VMEM((2 * D,), jnp.float32),
    ],
)(_sc_body)


def kernel(global_proj, player_proj, hex_proj, pos_global, pos_player, W_x, W_y):
    og2, op2 = _sc_call(global_proj.reshape(B, D),
                        player_proj.reshape(2 * B, D),
                        pos_global.reshape(D),
                        pos_player.reshape(2 * D))

    ht = hex_proj.transpose(1, 0, 2)
    h_t = pl.pallas_call(
        _tc_body,
        grid=(ROWS,),
        in_specs=[
            pl.BlockSpec(memory_space=pl.ANY),
            pl.BlockSpec((15, D), lambda i: (0, 0)),
            pl.BlockSpec((11, D), lambda i: (0, 0)),
        ],
        out_specs=pl.BlockSpec(memory_space=pl.ANY),
        out_shape=jax.ShapeDtypeStruct((ROWS, B, D), jnp.float32),
        scratch_shapes=[
            pltpu.VMEM((NBUF, B, D), jnp.float32),
            pltpu.VMEM((NBUF, B, D), jnp.float32),
            pltpu.VMEM((ROWS, D), jnp.float32),
            pltpu.SemaphoreType.DMA((NBUF,)),
            pltpu.SemaphoreType.DMA((NBUF,)),
        ],
    )(ht, W_x, W_y)

    return (og2.reshape(B, 1, D), op2.reshape(B, 2, D), h_t.transpose(1, 0, 2))


# speedup vs baseline: 1.0736x; 1.0736x over previous
import jax
import jax.numpy as jnp
from jax.experimental import pallas as pl
from jax.experimental.pallas import tpu as pltpu

B = 4096
D = 128
ROWS = 165
NBUF = 11


def _body(g_hbm, p_hbm, h_hbm, pg_ref, pp_ref, wx_ref, wy_ref,
          og_hbm, op_hbm, oh_hbm,
          inb, outb, gbuf, pbuf, gob, pob, pe_ref,
          in_sems, out_sems, gp_sems):
    i = pl.program_id(0)

    def in_copy(chunk, slot):
        return pltpu.make_async_copy(h_hbm.at[chunk], inb.at[slot],
                                     in_sems.at[slot])

    def out_copy(chunk, slot):
        return pltpu.make_async_copy(outb.at[slot], oh_hbm.at[chunk],
                                     out_sems.at[slot])

    g_in = pltpu.make_async_copy(g_hbm, gbuf, gp_sems.at[0])
    p_in = pltpu.make_async_copy(p_hbm, pbuf, gp_sems.at[1])
    g_out = pltpu.make_async_copy(gob, og_hbm, gp_sems.at[2])
    p_out = pltpu.make_async_copy(pob, op_hbm, gp_sems.at[3])

    @pl.when(i == 0)
    def _prime():
        wx = wx_ref[...]
        for y in range(11):
            pe_ref[pl.ds(15 * y, 15), :] = wy_ref[y:y + 1, :] + wx
        g_in.start()
        p_in.start()
        for k in range(NBUF):
            in_copy(k, k).start()

    @pl.when(i == 1)
    def _do_g():
        g_in.wait()
        gob[...] = gbuf[...] + pg_ref[...]
        g_out.start()

    @pl.when(i == 2)
    def _do_p():
        p_in.wait()
        pob[...] = pbuf[...] + pp_ref[...]
        p_out.start()

    s = jax.lax.rem(i, NBUF)
    in_copy(i, s).wait()

    @pl.when(i >= NBUF)
    def _wait_out():
        out_copy(i, s).wait()

    outb[s] = inb[s] + pe_ref[pl.ds(i, 1), :]
    out_copy(i, s).start()

    @pl.when(i + NBUF < ROWS)
    def _next_in():
        in_copy(i + NBUF, s).start()

    @pl.when(i == ROWS - 1)
    def _drain():
        for k in range(NBUF):
            out_copy(0, k).wait()
        g_out.wait()
        p_out.wait()


def kernel(global_proj, player_proj, hex_proj, pos_global, pos_player, W_x, W_y):
    ht = hex_proj.transpose(1, 0, 2)
    out = pl.pallas_call(
        _body,
        grid=(ROWS,),
        in_specs=[
            pl.BlockSpec(memory_space=pl.ANY),
            pl.BlockSpec(memory_space=pl.ANY),
            pl.BlockSpec(memory_space=pl.ANY),
            pl.BlockSpec((1, D), lambda i: (0, 0)),
            pl.BlockSpec((2, D), lambda i: (0, 0)),
            pl.BlockSpec((15, D), lambda i: (0, 0)),
            pl.BlockSpec((11, D), lambda i: (0, 0)),
        ],
        out_specs=[
            pl.BlockSpec(memory_space=pl.ANY),
            pl.BlockSpec(memory_space=pl.ANY),
            pl.BlockSpec(memory_space=pl.ANY),
        ],
        out_shape=[
            jax.ShapeDtypeStruct((B, 1, D), jnp.float32),
            jax.ShapeDtypeStruct((B, 2, D), jnp.float32),
            jax.ShapeDtypeStruct((ROWS, B, D), jnp.float32),
        ],
        scratch_shapes=[
            pltpu.VMEM((NBUF, B, D), jnp.float32),
            pltpu.VMEM((NBUF, B, D), jnp.float32),
            pltpu.VMEM((B, 1, D), jnp.float32),
            pltpu.VMEM((B, 2, D), jnp.float32),
            pltpu.VMEM((B, 1, D), jnp.float32),
            pltpu.VMEM((B, 2, D), jnp.float32),
            pltpu.VMEM((ROWS, D), jnp.float32),
            pltpu.SemaphoreType.DMA((NBUF,)),
            pltpu.SemaphoreType.DMA((NBUF,)),
            pltpu.SemaphoreType.DMA((4,)),
        ],
    )(global_proj, player_proj, ht, pos_global, pos_player, W_x, W_y)
    g, p, h_t = out
    return (g, p, h_t.transpose(1, 0, 2))


# speedup vs baseline: 1.0742x; 1.0006x over previous
import jax
import jax.numpy as jnp
from jax.experimental import pallas as pl
from jax.experimental.pallas import tpu as pltpu

B = 4096
D = 128
ROWS = 165
NBUF = 10


def _body(g_hbm, p_hbm, h_hbm, pg_ref, pp_ref, wx_ref, wy_ref,
          og_hbm, op_hbm, oh_hbm,
          inb, outb, gbuf, pbuf, gob, pob, pe_ref,
          in_sems, out_sems, gp_sems):
    i = pl.program_id(0)

    def in_copy(chunk, slot):
        return pltpu.make_async_copy(h_hbm.at[chunk], inb.at[slot],
                                     in_sems.at[slot])

    def out_copy(chunk, slot):
        return pltpu.make_async_copy(outb.at[slot], oh_hbm.at[chunk],
                                     out_sems.at[slot])

    g_in = pltpu.make_async_copy(g_hbm, gbuf, gp_sems.at[0])
    p_in = pltpu.make_async_copy(p_hbm, pbuf, gp_sems.at[1])
    g_out = pltpu.make_async_copy(gob, og_hbm, gp_sems.at[2])
    p_out = pltpu.make_async_copy(pob, op_hbm, gp_sems.at[3])

    @pl.when(i == 0)
    def _prime():
        wx = wx_ref[...]
        for y in range(11):
            pe_ref[pl.ds(15 * y, 15), :] = wy_ref[y:y + 1, :] + wx
        g_in.start()
        p_in.start()
        for k in range(NBUF):
            in_copy(k, k).start()

    @pl.when(i == 1)
    def _do_g():
        g_in.wait()
        gob[...] = gbuf[...] + pg_ref[...]
        g_out.start()

    @pl.when(i == 2)
    def _do_p():
        p_in.wait()
        pob[...] = pbuf[...] + pp_ref[...]
        p_out.start()

    s = jax.lax.rem(i, NBUF)
    in_copy(i, s).wait()

    @pl.when(i >= NBUF)
    def _wait_out():
        out_copy(i, s).wait()

    outb[s] = inb[s] + pe_ref[pl.ds(i, 1), :]
    out_copy(i, s).start()

    @pl.when(i + NBUF < ROWS)
    def _next_in():
        in_copy(i + NBUF, s).start()

    @pl.when(i == ROWS - 1)
    def _drain():
        for k in range(NBUF):
            out_copy(0, k).wait()
        g_out.wait()
        p_out.wait()


def kernel(global_proj, player_proj, hex_proj, pos_global, pos_player, W_x, W_y):
    ht = hex_proj.transpose(1, 0, 2)
    out = pl.pallas_call(
        _body,
        grid=(ROWS,),
        in_specs=[
            pl.BlockSpec(memory_space=pl.ANY),
            pl.BlockSpec(memory_space=pl.ANY),
            pl.BlockSpec(memory_space=pl.ANY),
            pl.BlockSpec((1, D), lambda i: (0, 0)),
            pl.BlockSpec((2, D), lambda i: (0, 0)),
            pl.BlockSpec((15, D), lambda i: (0, 0)),
            pl.BlockSpec((11, D), lambda i: (0, 0)),
        ],
        out_specs=[
            pl.BlockSpec(memory_space=pl.ANY),
            pl.BlockSpec(memory_space=pl.ANY),
            pl.BlockSpec(memory_space=pl.ANY),
        ],
        out_shape=[
            jax.ShapeDtypeStruct((B, 1, D), jnp.float32),
            jax.ShapeDtypeStruct((B, 2, D), jnp.float32),
            jax.ShapeDtypeStruct((ROWS, B, D), jnp.float32),
        ],
        scratch_shapes=[
            pltpu.VMEM((NBUF, B, D), jnp.float32),
            pltpu.VMEM((NBUF, B, D), jnp.float32),
            pltpu.VMEM((B, 1, D), jnp.float32),
            pltpu.VMEM((B, 2, D), jnp.float32),
            pltpu.VMEM((B, 1, D), jnp.float32),
            pltpu.VMEM((B, 2, D), jnp.float32),
            pltpu.VMEM((ROWS, D), jnp.float32),
            pltpu.SemaphoreType.DMA((NBUF,)),
            pltpu.SemaphoreType.DMA((NBUF,)),
            pltpu.SemaphoreType.DMA((4,)),
        ],
    )(global_proj, player_proj, ht, pos_global, pos_player, W_x, W_y)
    g, p, h_t = out
    return (g, p, h_t.transpose(1, 0, 2))
